# initial kernel scaffold (unmeasured)
import jax
import jax.numpy as jnp
from jax import lax
from jax.experimental import pallas as pl
from jax.experimental.pallas import tpu as pltpu

N_DEV = 4
SQ = 512
SKV = 2048
HQ = 8
DH = 128
DM = 1024
SCALE = 0.08838834764831843


def _body(x_ref, wq_ref, wo_ref, kv_ref, out_ref, comm_ref, send_sems, recv_sems):
    my = lax.axis_index("i")
    left = (my + N_DEV - 1) % N_DEV
    right = (my + 1) % N_DEV

    barrier_sem = pltpu.get_barrier_semaphore()
    for nbr in (left, right):
        pl.semaphore_signal(
            barrier_sem, inc=1,
            device_id=(nbr,), device_id_type=pl.DeviceIdType.MESH,
        )
    pl.semaphore_wait(barrier_sem, 2)

    def start_hop(h):
        src = kv_ref if h == 0 else comm_ref.at[h - 1]
        rdma = pltpu.make_async_remote_copy(
            src_ref=src,
            dst_ref=comm_ref.at[h],
            send_sem=send_sems.at[h],
            recv_sem=recv_sems.at[h],
            device_id=(right,),
            device_id_type=pl.DeviceIdType.MESH,
        )
        rdma.start()
        return rdma

    q = lax.dot_general(
        x_ref[...], wq_ref[...], (((1,), (0,)), ((), ())),
        preferred_element_type=jnp.float32,
    )
    q = (q * SCALE).astype(jnp.bfloat16)

    l = [None] * HQ
    acc = [None] * HQ

    def process(k_of, v_of, first):
        for h in range(HQ):
            k = k_of(h)
            v = v_of(h)
            qh = q[:, h * DH:(h + 1) * DH]
            s = lax.dot_general(
                qh, k, (((1,), (1,)), ((), ())),
                preferred_element_type=jnp.float32,
            )
            p = jnp.exp(s)
            pv = lax.dot_general(
                p.astype(jnp.bfloat16), v, (((1,), (0,)), ((), ())),
                preferred_element_type=jnp.float32,
            )
            ps = jnp.sum(p, axis=1, keepdims=True)
            if first:
                l[h] = ps
                acc[h] = pv
            else:
                l[h] = l[h] + ps
                acc[h] = acc[h] + pv

    for hop in range(N_DEV - 1):
        rdma = start_hop(hop)
        if hop == 0:
            process(lambda hd: kv_ref[0, hd], lambda hd: kv_ref[1, hd], True)
        else:
            process(
                lambda hd: comm_ref[hop - 1, 0, hd],
                lambda hd: comm_ref[hop - 1, 1, hd],
                False,
            )
        rdma.wait()
    process(
        lambda hd: comm_ref[N_DEV - 2, 0, hd],
        lambda hd: comm_ref[N_DEV - 2, 1, hd],
        False,
    )

    out = None
    for h in range(HQ):
        o = (acc[h] / l[h]).astype(jnp.bfloat16)
        contrib = lax.dot_general(
            o, wo_ref[h * DH:(h + 1) * DH, :], (((1,), (0,)), ((), ())),
            preferred_element_type=jnp.float32,
        )
        out = contrib if out is None else out + contrib
    out_ref[...] = out


def kernel(x, Wq, Wo, K_ext, V_ext):
    xb = x[0].astype(jnp.bfloat16)
    wqb = Wq.astype(jnp.bfloat16)
    wob = Wo.astype(jnp.bfloat16)
    kv = jnp.stack([K_ext[0], V_ext[0]]).astype(jnp.bfloat16)
    kv = kv.transpose(0, 2, 1, 3)

    out = pl.pallas_call(
        _body,
        out_shape=jax.ShapeDtypeStruct((SQ, DM), jnp.float32),
        in_specs=[pl.BlockSpec(memory_space=pltpu.VMEM)] * 4,
        out_specs=pl.BlockSpec(memory_space=pltpu.VMEM),
        scratch_shapes=[
            pltpu.VMEM((N_DEV - 1, 2, HQ, SKV, DH), jnp.bfloat16),
            pltpu.SemaphoreType.DMA((N_DEV - 1,)),
            pltpu.SemaphoreType.DMA((N_DEV - 1,)),
        ],
        compiler_params=pltpu.CompilerParams(collective_id=0),
    )(xb, wqb, wob, kv)
    return out[None]


# baseline (device time: 316086 ns/iter reference)
import jax
import jax.numpy as jnp
from jax import lax
from jax.experimental import pallas as pl
from jax.experimental.pallas import tpu as pltpu

N_DEV = 4
SQ = 512
SKV = 2048
TKV = 1024
HQ = 8
DH = 128
DM = 1024
SCALE = 0.08838834764831843


def _body(x_ref, wq_ref, wo_ref, kv_ref, out_ref,
          comm_ref, acc_ref, send_sems, recv_sems, credit_sem):
    my = lax.axis_index("i")
    left = (my + N_DEV - 1) % N_DEV
    right = (my + 1) % N_DEV

    barrier_sem = pltpu.get_barrier_semaphore()
    for nbr in (left, right):
        pl.semaphore_signal(
            barrier_sem, inc=1,
            device_id=(nbr,), device_id_type=pl.DeviceIdType.MESH,
        )
    pl.semaphore_wait(barrier_sem, 2)

    def start_hop(h):
        src = kv_ref if h == 0 else comm_ref.at[(h - 1) % 2]
        rdma = pltpu.make_async_remote_copy(
            src_ref=src,
            dst_ref=comm_ref.at[h % 2],
            send_sem=send_sems.at[h],
            recv_sem=recv_sems.at[h],
            device_id=(right,),
            device_id_type=pl.DeviceIdType.MESH,
        )
        rdma.start()
        return rdma

    q = lax.dot_general(
        x_ref[...], wq_ref[...], (((1,), (0,)), ((), ())),
        preferred_element_type=jnp.float32,
    )
    q = (q * SCALE).astype(jnp.bfloat16)

    l = [None] * HQ

    def process(ref, k_idx, v_idx, first):
        for h in range(HQ):
            qh = q[:, h * DH:(h + 1) * DH]
            for t in range(SKV // TKV):
                tsl = pl.ds(t * TKV, TKV)
                k = ref[k_idx, h, tsl, :]
                v = ref[v_idx, h, tsl, :]
                s = lax.dot_general(
                    qh, k, (((1,), (1,)), ((), ())),
                    preferred_element_type=jnp.float32,
                )
                p = jnp.exp(s)
                pv = lax.dot_general(
                    p.astype(jnp.bfloat16), v, (((1,), (0,)), ((), ())),
                    preferred_element_type=jnp.float32,
                )
                ps = jnp.sum(p, axis=1, keepdims=True)
                if first and t == 0:
                    acc_ref[h] = pv
                    l[h] = ps
                else:
                    acc_ref[h] = acc_ref[h] + pv
                    l[h] = l[h] + ps

    for hop in range(N_DEV - 1):
        rdma = start_hop(hop)
        if hop == 0:
            process(kv_ref, 0, 1, True)
        else:
            slot = (hop - 1) % 2
            process(comm_ref.at[slot], 0, 1, False)
            if hop == 1:
                pl.semaphore_signal(
                    credit_sem, inc=1,
                    device_id=(left,), device_id_type=pl.DeviceIdType.MESH,
                )
        if hop == 1:
            pl.semaphore_wait(credit_sem, 1)
        rdma.wait()
    process(comm_ref.at[(N_DEV - 2) % 2], 0, 1, False)

    out = None
    for h in range(HQ):
        o = (acc_ref[h] / l[h]).astype(jnp.bfloat16)
        contrib = lax.dot_general(
            o, wo_ref[h * DH:(h + 1) * DH, :], (((1,), (0,)), ((), ())),
            preferred_element_type=jnp.float32,
        )
        out = contrib if out is None else out + contrib
    out_ref[...] = out


def kernel(x, Wq, Wo, K_ext, V_ext):
    xb = x[0].astype(jnp.bfloat16)
    wqb = Wq.astype(jnp.bfloat16)
    wob = Wo.astype(jnp.bfloat16)
    kv = jnp.stack([K_ext[0], V_ext[0]]).astype(jnp.bfloat16)
    kv = kv.transpose(0, 2, 1, 3)

    out = pl.pallas_call(
        _body,
        out_shape=jax.ShapeDtypeStruct((SQ, DM), jnp.float32),
        in_specs=[pl.BlockSpec(memory_space=pltpu.VMEM)] * 4,
        out_specs=pl.BlockSpec(memory_space=pltpu.VMEM),
        scratch_shapes=[
            pltpu.VMEM((2, 2, HQ, SKV, DH), jnp.bfloat16),
            pltpu.VMEM((HQ, SQ, DH), jnp.float32),
            pltpu.SemaphoreType.DMA((N_DEV - 1,)),
            pltpu.SemaphoreType.DMA((N_DEV - 1,)),
            pltpu.SemaphoreType.REGULAR,
        ],
        compiler_params=pltpu.CompilerParams(
            collective_id=0, vmem_limit_bytes=50 * 1024 * 1024,
        ),
    )(xb, wqb, wob, kv)
    return out[None]


# device time: 138867 ns/iter; 2.2762x vs baseline; 2.2762x over previous
import jax
import jax.numpy as jnp
from jax import lax
from jax.experimental import pallas as pl
from jax.experimental.pallas import tpu as pltpu

N_DEV = 4
SQ = 512
SKV = 2048
TKV = 1024
HQ = 8
DH = 128
DM = 1024
SCALE = 0.08838834764831843


def _body(x_ref, wq_ref, wo_ref, kv_ref, out_ref,
          xag_ref, pacc_ref, sacc_ref, racc_ref, sl_ref, rl_ref,
          ag_ss, ag_rs, acc_ss, acc_rs, l_ss, l_rs):
    my = lax.axis_index("i")
    left = (my + N_DEV - 1) % N_DEV
    right = (my + 1) % N_DEV

    barrier_sem = pltpu.get_barrier_semaphore()
    for nbr in (left, right):
        pl.semaphore_signal(
            barrier_sem, inc=1,
            device_id=(nbr,), device_id_type=pl.DeviceIdType.MESH,
        )
    pl.semaphore_wait(barrier_sem, 2)

    def remote_copy(src, dst, ssem, rsem):
        rdma = pltpu.make_async_remote_copy(
            src_ref=src, dst_ref=dst, send_sem=ssem, recv_sem=rsem,
            device_id=(right,), device_id_type=pl.DeviceIdType.MESH,
        )
        rdma.start()
        return rdma

    def proj_q(xv):
        q = lax.dot_general(
            xv, wq_ref[...], (((1,), (0,)), ((), ())),
            preferred_element_type=jnp.float32,
        )
        return (q * SCALE).astype(jnp.bfloat16)

    def partial(q):
        ls = []
        for h in range(HQ):
            qh = q[:, h * DH:(h + 1) * DH]
            lh = None
            for t in range(SKV // TKV):
                tsl = pl.ds(t * TKV, TKV)
                k = kv_ref[0, h, tsl, :]
                v = kv_ref[1, h, tsl, :]
                s = lax.dot_general(
                    qh, k, (((1,), (1,)), ((), ())),
                    preferred_element_type=jnp.float32,
                )
                p = jnp.exp(s)
                pv = lax.dot_general(
                    p.astype(jnp.bfloat16), v, (((1,), (0,)), ((), ())),
                    preferred_element_type=jnp.float32,
                )
                ps = jnp.sum(p, axis=1)
                if t == 0:
                    pacc_ref[h] = pv
                    lh = ps
                else:
                    pacc_ref[h] = pacc_ref[h] + pv
                    lh = lh + ps
            ls.append(lh)
        return ls

    rdmas = []

    ag0 = remote_copy(x_ref, xag_ref.at[0], ag_ss.at[0], ag_rs.at[0])
    rdmas.append(ag0)
    q_own = proj_q(x_ref[...])

    ag0.wait_recv()
    ag1 = remote_copy(xag_ref.at[0], xag_ref.at[1], ag_ss.at[1], ag_rs.at[1])
    rdmas.append(ag1)

    l_cur = partial(proj_q(xag_ref[0]))
    for h in range(HQ):
        sacc_ref[0, h] = pacc_ref[h]
        sl_ref[0, h] = l_cur[h]
    rs0a = remote_copy(sacc_ref.at[0], racc_ref.at[0], acc_ss.at[0], acc_rs.at[0])
    rs0l = remote_copy(sl_ref.at[0], rl_ref.at[0], l_ss.at[0], l_rs.at[0])
    rdmas += [rs0a, rs0l]

    ag1.wait_recv()
    ag2 = remote_copy(xag_ref.at[1], xag_ref.at[2], ag_ss.at[2], ag_rs.at[2])
    rdmas.append(ag2)

    l_cur = partial(proj_q(xag_ref[1]))
    rs0a.wait_recv()
    rs0l.wait_recv()
    for h in range(HQ):
        sacc_ref[1, h] = pacc_ref[h] + racc_ref[0, h]
        sl_ref[1, h] = l_cur[h] + rl_ref[0, h]
    rs1a = remote_copy(sacc_ref.at[1], racc_ref.at[1], acc_ss.at[1], acc_rs.at[1])
    rs1l = remote_copy(sl_ref.at[1], rl_ref.at[1], l_ss.at[1], l_rs.at[1])
    rdmas += [rs1a, rs1l]

    ag2.wait_recv()

    l_cur = partial(proj_q(xag_ref[2]))
    rs1a.wait_recv()
    rs1l.wait_recv()
    for h in range(HQ):
        sacc_ref[2, h] = pacc_ref[h] + racc_ref[1, h]
        sl_ref[2, h] = l_cur[h] + rl_ref[1, h]
    rs2a = remote_copy(sacc_ref.at[2], racc_ref.at[2], acc_ss.at[2], acc_rs.at[2])
    rs2l = remote_copy(sl_ref.at[2], rl_ref.at[2], l_ss.at[2], l_rs.at[2])
    rdmas += [rs2a, rs2l]

    l_cur = partial(q_own)
    rs2a.wait_recv()
    rs2l.wait_recv()

    out = None
    for h in range(HQ):
        acc = pacc_ref[h] + racc_ref[2, h]
        lh = l_cur[h] + rl_ref[2, h]
        o = (acc / lh[:, None]).astype(jnp.bfloat16)
        contrib = lax.dot_general(
            o, wo_ref[h * DH:(h + 1) * DH, :], (((1,), (0,)), ((), ())),
            preferred_element_type=jnp.float32,
        )
        out = contrib if out is None else out + contrib
    out_ref[...] = out

    for r in rdmas:
        r.wait_send()


def kernel(x, Wq, Wo, K_ext, V_ext):
    xb = x[0].astype(jnp.bfloat16)
    wqb = Wq.astype(jnp.bfloat16)
    wob = Wo.astype(jnp.bfloat16)
    kv = jnp.stack([K_ext[0], V_ext[0]]).astype(jnp.bfloat16)
    kv = kv.transpose(0, 2, 1, 3)

    out = pl.pallas_call(
        _body,
        out_shape=jax.ShapeDtypeStruct((SQ, DM), jnp.float32),
        in_specs=[pl.BlockSpec(memory_space=pltpu.VMEM)] * 4,
        out_specs=pl.BlockSpec(memory_space=pltpu.VMEM),
        scratch_shapes=[
            pltpu.VMEM((N_DEV - 1, SQ, DM), jnp.bfloat16),
            pltpu.VMEM((HQ, SQ, DH), jnp.float32),
            pltpu.VMEM((N_DEV - 1, HQ, SQ, DH), jnp.float32),
            pltpu.VMEM((N_DEV - 1, HQ, SQ, DH), jnp.float32),
            pltpu.VMEM((N_DEV - 1, HQ, SQ), jnp.float32),
            pltpu.VMEM((N_DEV - 1, HQ, SQ), jnp.float32),
            pltpu.SemaphoreType.DMA((N_DEV - 1,)),
            pltpu.SemaphoreType.DMA((N_DEV - 1,)),
            pltpu.SemaphoreType.DMA((N_DEV - 1,)),
            pltpu.SemaphoreType.DMA((N_DEV - 1,)),
            pltpu.SemaphoreType.DMA((N_DEV - 1,)),
            pltpu.SemaphoreType.DMA((N_DEV - 1,)),
        ],
        compiler_params=pltpu.CompilerParams(
            collective_id=0, vmem_limit_bytes=50 * 1024 * 1024,
        ),
    )(xb, wqb, wob, kv)
    return out[None]
